# Initial kernel scaffold; baseline (speedup 1.0000x reference)
import jax
import jax.numpy as jnp
from jax.experimental import pallas as pl


def kernel(inp_patch, label_loc):
    raise NotImplementedError("write your pallas kernel here")


# speedup vs baseline: 1.7336x; 1.7336x over previous
import functools

import jax
import jax.numpy as jnp
from jax import lax
from jax.experimental import pallas as pl
from jax.experimental.pallas import tpu as pltpu
from jax.experimental.pallas import tpu_sc as plsc

OUT_SZ = 256
IN_SZ = 1024
HALF = OUT_SZ // 2
BATCH = 32
N_TOTAL = BATCH * IN_SZ * IN_SZ
TROWS = N_TOTAL // OUT_SZ
CHUNK = 64
NCHUNK = OUT_SZ // CHUNK
NLANE = 16


def _sc_crop_gather(table, idx, offs):
    mesh = plsc.VectorSubcoreMesh(core_axis_name="c", subcore_axis_name="s")

    @functools.partial(
        pl.kernel,
        out_type=jax.ShapeDtypeStruct((BATCH * OUT_SZ, OUT_SZ), jnp.float32),
        mesh=mesh,
        compiler_params=pltpu.CompilerParams(
            use_tc_tiling_on_sc=False, needs_layout_passes=False),
        scratch_types=[
            pltpu.VMEM((NCHUNK, 2 * CHUNK), jnp.int32),
            pltpu.VMEM((NLANE,), jnp.int32),
            pltpu.VMEM((2 * CHUNK, OUT_SZ), jnp.float32),
            pltpu.VMEM((CHUNK, OUT_SZ), jnp.float32),
            pltpu.SemaphoreType.DMA,
        ],
    )
    def k(table_hbm, idx_hbm, offs_hbm, out_hbm, idx_v, off_v, rows_v, outb_v, gsem):
        w = lax.axis_index("s") * 2 + lax.axis_index("c")
        pltpu.sync_copy(idx_hbm.at[w], idx_v)
        pltpu.sync_copy(offs_hbm.at[w], off_v)
        lanes = lax.iota(jnp.int32, NLANE)
        base16 = off_v[...] + lanes
        cols = [lanes + kk * NLANE for kk in range(OUT_SZ // NLANE)]

        for c in range(NCHUNK):
            pltpu.async_copy(table_hbm.at[idx_v.at[c]], rows_v, gsem).wait()

            def realign(u, _):
                vecbase = base16 + u * (2 * OUT_SZ)
                urow = jnp.full((NLANE,), u, dtype=jnp.int32)
                for kk in range(OUT_SZ // NLANE):
                    p = vecbase + kk * NLANE
                    v = plsc.load_gather(rows_v, [p >> 8, p & 255])
                    plsc.store_scatter(outb_v, [urow, cols[kk]], v)
                return 0

            lax.fori_loop(0, CHUNK, realign, 0)
            pltpu.sync_copy(outb_v, out_hbm.at[pl.ds(w * OUT_SZ + c * CHUNK, CHUNK)])

    return k(table, idx, offs)


def kernel(inp_patch, label_loc):
    nbatch, nch, nr, nc = inp_patch.shape
    frame_start = label_loc.astype(jnp.int32) - HALF
    fx = frame_start[:, 0]
    fy = frame_start[:, 1]
    b = jnp.arange(BATCH, dtype=jnp.int32)
    s0 = b * (IN_SZ * IN_SZ) + fy * IN_SZ + fx
    yi = jnp.arange(OUT_SZ, dtype=jnp.int32)
    s = s0[:, None] + yi[None, :] * IN_SZ
    smod = jnp.mod(s, N_TOTAL)
    r0 = smod // OUT_SZ
    idx = jnp.stack([r0, jnp.mod(r0 + 1, TROWS)], axis=-1)
    idx = idx.reshape(BATCH, NCHUNK, 2 * CHUNK).astype(jnp.int32)
    off = (smod[:, :1] % OUT_SZ).astype(jnp.int32)
    offs = jnp.broadcast_to(off, (BATCH, NLANE))

    table = inp_patch.reshape(TROWS, OUT_SZ)
    out = _sc_crop_gather(table, idx, offs)
    out_patch = out.reshape(nbatch, nch, OUT_SZ, OUT_SZ)

    new_label = (label_loc - frame_start.astype(jnp.float32)) / OUT_SZ
    return out_patch, new_label.astype(jnp.float32)
